# baseline (speedup 1.0000x reference)
import jax
import jax.numpy as jnp
from jax.experimental import pallas as pl
from jax.experimental.pallas import tpu as pltpu

_SIZE = 65536
_SHIFT = 8192
_ROWS = 64
_RB = 8
_NS = _ROWS // _RB


def _body(x_ref, o_ref):
    o_ref[:, : _SIZE - _SHIFT] = jnp.zeros(
        (_RB, _SIZE - _SHIFT), jnp.float32)
    o_ref[:, _SIZE - _SHIFT :] = x_ref[...]


def kernel(x):
    xf = x.reshape(_ROWS, _SHIFT)
    out = pl.pallas_call(
        _body,
        grid=(_NS,),
        in_specs=[pl.BlockSpec((_RB, _SHIFT), lambda i: (i, 0))],
        out_specs=pl.BlockSpec((_RB, _SIZE), lambda i: (i, 0)),
        out_shape=jax.ShapeDtypeStruct((_ROWS, _SIZE), jnp.float32),
        compiler_params=pltpu.CompilerParams(
            dimension_semantics=("arbitrary",),
        ),
    )(xf)
    return out.reshape(x.shape[:-1] + (_SIZE,))


# speedup vs baseline: 1.0562x; 1.0562x over previous
import jax
import jax.numpy as jnp
from jax.experimental import pallas as pl
from jax.experimental.pallas import tpu as pltpu

_SIZE = 65536
_SHIFT = 8192
_ZCOLS = _SIZE - _SHIFT
_ROWS = 64
_RB = 8
_NS = _ROWS // _RB


def _body(x_hbm, o_hbm, zbuf, xbuf, zsems, isems, osems):
    xins = [
        pltpu.make_async_copy(
            x_hbm.at[pl.ds(s * _RB, _RB), :], xbuf.at[s], isems.at[s])
        for s in range(_NS)
    ]
    for c in xins:
        c.start()

    zbuf[...] = jnp.zeros((_RB, _ZCOLS), jnp.float32)

    zcps = [
        pltpu.make_async_copy(
            zbuf, o_hbm.at[pl.ds(s * _RB, _RB), pl.ds(0, _ZCOLS)],
            zsems.at[s])
        for s in range(_NS)
    ]
    for c in zcps:
        c.start()

    xouts = []
    for s in range(_NS):
        xins[s].wait()
        c = pltpu.make_async_copy(
            xbuf.at[s], o_hbm.at[pl.ds(s * _RB, _RB), pl.ds(_ZCOLS, _SHIFT)],
            osems.at[s])
        c.start()
        xouts.append(c)
    for c in zcps:
        c.wait()
    for c in xouts:
        c.wait()


def kernel(x):
    xf = x.reshape(_ROWS, _SHIFT)
    out = pl.pallas_call(
        _body,
        in_specs=[pl.BlockSpec(memory_space=pl.ANY)],
        out_specs=pl.BlockSpec(memory_space=pl.ANY),
        out_shape=jax.ShapeDtypeStruct((_ROWS, _SIZE), jnp.float32),
        scratch_shapes=[
            pltpu.VMEM((_RB, _ZCOLS), jnp.float32),
            pltpu.VMEM((_NS, _RB, _SHIFT), jnp.float32),
            pltpu.SemaphoreType.DMA((_NS,)),
            pltpu.SemaphoreType.DMA((_NS,)),
            pltpu.SemaphoreType.DMA((_NS,)),
        ],
    )(xf)
    return out.reshape(x.shape[:-1] + (_SIZE,))
